# baseline (speedup 1.0000x reference)
import functools

import jax
import jax.numpy as jnp
from jax import lax
from jax.experimental import pallas as pl
from jax.experimental.pallas import tpu as pltpu
from jax.experimental.pallas import tpu_sc as plsc

_RB = 128
_JC = 256
_BIG = 1 << 20


def _sel_body(n_total, n_batches, pcs_ref, pcsT_ref, sel_ref):
    bidx = pl.program_id(0)
    nblk = pl.program_id(1)
    n0 = nblk * _RB

    cx = pcsT_ref[0, :, 0:1]
    cy = pcsT_ref[0, :, 1:2]
    cz = pcsT_ref[0, :, 2:3]
    sqc = cx * cx + cy * cy + cz * cz
    cxb = cx.astype(jnp.bfloat16).astype(jnp.float32) * -2.0
    cyb = cy.astype(jnp.bfloat16).astype(jnp.float32) * -2.0
    czb = cz.astype(jnp.bfloat16).astype(jnp.float32) * -2.0

    rowi = lax.broadcasted_iota(jnp.int32, (_RB, 1), 0)
    nvec = n0 + rowi
    coli = lax.broadcasted_iota(jnp.int32, (_RB, _JC), 1)
    tri = (lax.broadcasted_iota(jnp.int32, (_JC, _JC), 0)
           < lax.broadcasted_iota(jnp.int32, (_JC, _JC), 1)).astype(jnp.float32)

    cnt = jnp.zeros((_RB, 1), jnp.float32)
    best = [jnp.full((_RB, _JC), _BIG, jnp.int32) for _ in range(7)]
    for ci in range(n_total // _JC):
        j0 = ci * _JC
        jx = pcs_ref[0, 0:1, j0:j0 + _JC]
        jy = pcs_ref[0, 1:2, j0:j0 + _JC]
        jz = pcs_ref[0, 2:3, j0:j0 + _JC]
        dx = jx - cx
        dy = jy - cy
        dz = jz - cz
        sqj = jx * jx + jy * jy + jz * jz
        jxb = jx.astype(jnp.bfloat16).astype(jnp.float32)
        jyb = jy.astype(jnp.bfloat16).astype(jnp.float32)
        jzb = jz.astype(jnp.bfloat16).astype(jnp.float32)
        inner2 = cxb * jxb + cyb * jyb + czb * jzb
        d2 = (sqc + sqj) + inner2
        jglob = j0 + coli
        mask = (d2 < (0.2 * 0.2)) & (jglob != nvec)
        mf = mask.astype(jnp.float32)
        rank = jax.lax.dot(mf, tri, preferred_element_type=jnp.float32) + cnt
        elig = mask & (rank < 31.0)
        octv = ((dx > 0).astype(jnp.int32) * 4
                + (dy > 0).astype(jnp.int32) * 2
                + (dz > 0).astype(jnp.int32))
        jelig = jnp.where(elig, jglob, _BIG)
        for o in range(1, 8):
            key = jnp.where(octv == o, jelig, _BIG)
            best[o - 1] = jnp.minimum(best[o - 1], key)
        cnt = rank[:, _JC - 1:_JC] + mf[:, _JC - 1:_JC]

    bofs = bidx * n_total
    zero_row = n_batches * n_total
    cols = [jnp.broadcast_to(nvec + bofs, (_RB, 1))]
    for o in range(7):
        bo = jnp.min(best[o], axis=1, keepdims=True)
        cols.append(jnp.where(bo < _BIG, bo + bofs, zero_row))
    sel_ref[0, :, :] = jnp.concatenate(cols, axis=1)


def _sel_call(pcs, pcsT):
    b, _, n = pcs.shape
    return pl.pallas_call(
        functools.partial(_sel_body, n, b),
        grid=(b, n // _RB),
        in_specs=[
            pl.BlockSpec((1, 3, n), lambda bi, ni: (bi, 0, 0)),
            pl.BlockSpec((1, _RB, 3), lambda bi, ni: (bi, ni, 0)),
        ],
        out_specs=pl.BlockSpec((1, _RB, 8), lambda bi, ni: (bi, ni, 0)),
        out_shape=jax.ShapeDtypeStruct((b, n, 8), jnp.int32),
    )(pcs, pcsT)


_GCHUNK = 128


_NBUF = 4


def _gather_call(table, idx3):
    nw, nit, _ = idx3.shape
    rows = nw * nit * _GCHUNK
    ch = table.shape[1]
    info = plsc.get_sparse_core_info()
    nc = info.num_cores
    per_w = nit * _GCHUNK

    @functools.partial(
        pl.kernel,
        mesh=plsc.VectorSubcoreMesh(core_axis_name="c", subcore_axis_name="s"),
        out_type=jax.ShapeDtypeStruct((rows, ch), table.dtype),
        scratch_types=(
            [pltpu.VMEM((nit, _GCHUNK), jnp.int32),
             pltpu.VMEM_SHARED(table.shape, table.dtype)]
            + [pltpu.VMEM((_GCHUNK, ch), table.dtype) for _ in range(_NBUF)]
            + [pltpu.SemaphoreType.DMA for _ in range(_NBUF)]
        ),
    )
    def gk(table_hbm, idx_hbm, out_hbm, idx_v, table_sh, *bufs_sems):
        bufs = bufs_sems[:_NBUF]
        sems = bufs_sems[_NBUF:]
        sid = lax.axis_index("s")
        wid = sid * nc + lax.axis_index("c")
        base = wid * per_w
        @pl.when(sid == 0)
        def _():
            pltpu.sync_copy(table_hbm, table_sh)
        plsc.subcore_barrier()
        pltpu.sync_copy(idx_hbm.at[wid], idx_v)
        cps = [None] * _NBUF
        for i in range(min(_NBUF, nit)):
            cps[i] = pltpu.async_copy(
                table_sh.at[idx_v.at[i]], bufs[i], sems[i])
        for i in range(nit):
            k = i % _NBUF
            cps[k].wait()
            pltpu.sync_copy(bufs[k], out_hbm.at[pl.ds(base + i * _GCHUNK,
                                                      _GCHUNK)])
            nxt = i + _NBUF
            if nxt < nit:
                cps[k] = pltpu.async_copy(
                    table_sh.at[idx_v.at[nxt]], bufs[k], sems[k])

    return gk(table, idx3)


_MR = 256


def _conv_body(s_ref, w_ref, b_ref, o_ref):
    o_ref[...] = (
        jnp.dot(s_ref[...], w_ref[...], preferred_element_type=jnp.float32)
        + b_ref[...]
    )


def _conv_call(samples2d, w2d, b2d):
    rows, kc = samples2d.shape
    oc = w2d.shape[1]
    return pl.pallas_call(
        _conv_body,
        grid=(rows // _MR,),
        in_specs=[
            pl.BlockSpec((_MR, kc), lambda i: (i, 0)),
            pl.BlockSpec((kc, oc), lambda i: (0, 0)),
            pl.BlockSpec((1, oc), lambda i: (0, 0)),
        ],
        out_specs=pl.BlockSpec((_MR, oc), lambda i: (i, 0)),
        out_shape=jax.ShapeDtypeStruct((rows, oc), jnp.float32),
    )(samples2d, w2d, b2d)


def kernel(x, pcs, W, b):
    bb, c, n = x.shape
    oc = W.shape[0]

    pcsT = jnp.transpose(pcs, (0, 2, 1))
    xT = jnp.transpose(x, (0, 2, 1))
    info = plsc.get_sparse_core_info()
    nw = info.num_cores * info.num_subcores
    nit = (n * 8) // (nw * _GCHUNK)
    parts = []
    packed = lax.bitcast_convert_type(
        xT.astype(jnp.bfloat16).reshape(bb, n, c // 2, 2),
        jnp.int32)
    for bi in range(bb):
        sel = _sel_call(pcs[bi:bi + 1], pcsT[bi:bi + 1])
        table = jnp.concatenate(
            [packed[bi], jnp.zeros((1, c // 2), jnp.int32)], axis=0)
        parts.append(_gather_call(table, sel.reshape(nw, nit, _GCHUNK)))
    samples = lax.bitcast_convert_type(
        jnp.concatenate(parts, axis=0), jnp.bfloat16)

    w8 = jnp.concatenate([W[:, :, 0:1], W[:, :, 2:9]], axis=2)
    w2d = jnp.transpose(w8, (2, 1, 0)).reshape(8 * c, oc).astype(jnp.bfloat16)
    out2d = _conv_call(samples.reshape(bb * n, 8 * c), w2d,
                       b.reshape(1, oc))
    return out2d.reshape(bb, n, oc).transpose(0, 2, 1)


# speedup vs baseline: 12.9287x; 12.9287x over previous
import functools

import jax
import jax.numpy as jnp
from jax import lax
from jax.experimental import pallas as pl
from jax.experimental.pallas import tpu as pltpu
from jax.experimental.pallas import tpu_sc as plsc

_RB = 128
_JC = 256
_BIG = 1 << 20


def _sel_body(n_total, n_batches, pcs_ref, pcsT_ref, sel_ref):
    bidx = pl.program_id(0)
    nblk = pl.program_id(1)
    n0 = nblk * _RB

    cx = pcsT_ref[0, :, 0:1]
    cy = pcsT_ref[0, :, 1:2]
    cz = pcsT_ref[0, :, 2:3]
    sqc = cx * cx + cy * cy + cz * cz
    cxb = cx.astype(jnp.bfloat16).astype(jnp.float32) * -2.0
    cyb = cy.astype(jnp.bfloat16).astype(jnp.float32) * -2.0
    czb = cz.astype(jnp.bfloat16).astype(jnp.float32) * -2.0

    rowi = lax.broadcasted_iota(jnp.int32, (_RB, 1), 0)
    nvec = n0 + rowi
    coli = lax.broadcasted_iota(jnp.int32, (_RB, _JC), 1)
    tri = (lax.broadcasted_iota(jnp.int32, (_JC, _JC), 0)
           < lax.broadcasted_iota(jnp.int32, (_JC, _JC), 1)).astype(jnp.float32)

    cnt = jnp.zeros((_RB, 1), jnp.float32)
    best = [jnp.full((_RB, _JC), _BIG, jnp.int32) for _ in range(7)]
    for ci in range(n_total // _JC):
        j0 = ci * _JC
        jx = pcs_ref[0, 0:1, j0:j0 + _JC]
        jy = pcs_ref[0, 1:2, j0:j0 + _JC]
        jz = pcs_ref[0, 2:3, j0:j0 + _JC]
        dx = jx - cx
        dy = jy - cy
        dz = jz - cz
        sqj = jx * jx + jy * jy + jz * jz
        jxb = jx.astype(jnp.bfloat16).astype(jnp.float32)
        jyb = jy.astype(jnp.bfloat16).astype(jnp.float32)
        jzb = jz.astype(jnp.bfloat16).astype(jnp.float32)
        inner2 = cxb * jxb + cyb * jyb + czb * jzb
        d2 = (sqc + sqj) + inner2
        jglob = j0 + coli
        mask = (d2 < (0.2 * 0.2)) & (jglob != nvec)
        mf = mask.astype(jnp.float32)
        rank = jax.lax.dot(mf, tri, preferred_element_type=jnp.float32) + cnt
        elig = mask & (rank < 31.0)
        octv = ((dx > 0).astype(jnp.int32) * 4
                + (dy > 0).astype(jnp.int32) * 2
                + (dz > 0).astype(jnp.int32))
        jelig = jnp.where(elig, jglob, _BIG)
        for o in range(1, 8):
            key = jnp.where(octv == o, jelig, _BIG)
            best[o - 1] = jnp.minimum(best[o - 1], key)
        cnt = rank[:, _JC - 1:_JC] + mf[:, _JC - 1:_JC]

    bofs = bidx * n_total
    zero_row = n_batches * n_total
    cols = [jnp.broadcast_to(nvec + bofs, (_RB, 1))]
    for o in range(7):
        bo = jnp.min(best[o], axis=1, keepdims=True)
        cols.append(jnp.where(bo < _BIG, bo + bofs, zero_row))
    sel_ref[0, :, :] = jnp.concatenate(cols, axis=1)


def _sel_call(pcs, pcsT):
    b, _, n = pcs.shape
    return pl.pallas_call(
        functools.partial(_sel_body, n, b),
        grid=(b, n // _RB),
        in_specs=[
            pl.BlockSpec((1, 3, n), lambda bi, ni: (bi, 0, 0)),
            pl.BlockSpec((1, _RB, 3), lambda bi, ni: (bi, ni, 0)),
        ],
        out_specs=pl.BlockSpec((1, _RB, 8), lambda bi, ni: (bi, ni, 0)),
        out_shape=jax.ShapeDtypeStruct((b, n, 8), jnp.int32),
    )(pcs, pcsT)


_GCHUNK = 128


_NBUF = 4


def _gather_call(table, idx3):
    nw, nit, _ = idx3.shape
    rows = nw * nit * _GCHUNK
    ch = table.shape[1]
    info = plsc.get_sparse_core_info()
    nc = info.num_cores
    per_w = nit * _GCHUNK

    @functools.partial(
        pl.kernel,
        mesh=plsc.VectorSubcoreMesh(core_axis_name="c", subcore_axis_name="s"),
        out_type=jax.ShapeDtypeStruct((rows, ch), jnp.float32),
        scratch_types=(
            [pltpu.VMEM((nit, _GCHUNK), jnp.int32),
             pltpu.VMEM_SHARED(table.shape, jnp.float32)]
            + [pltpu.VMEM((_GCHUNK, ch), jnp.float32) for _ in range(_NBUF)]
            + [pltpu.SemaphoreType.DMA for _ in range(_NBUF)]
        ),
    )
    def gk(table_hbm, idx_hbm, out_hbm, idx_v, table_sh, *bufs_sems):
        bufs = bufs_sems[:_NBUF]
        sems = bufs_sems[_NBUF:]
        sid = lax.axis_index("s")
        wid = sid * nc + lax.axis_index("c")
        base = wid * per_w
        @pl.when(sid == 0)
        def _():
            pltpu.sync_copy(table_hbm, table_sh)
        plsc.subcore_barrier()
        pltpu.sync_copy(idx_hbm.at[wid], idx_v)
        cps = [None] * _NBUF
        for i in range(min(_NBUF, nit)):
            cps[i] = pltpu.async_copy(
                table_sh.at[idx_v.at[i]], bufs[i], sems[i])
        for i in range(nit):
            k = i % _NBUF
            cps[k].wait()
            pltpu.sync_copy(bufs[k], out_hbm.at[pl.ds(base + i * _GCHUNK,
                                                      _GCHUNK)])
            nxt = i + _NBUF
            if nxt < nit:
                cps[k] = pltpu.async_copy(
                    table_sh.at[idx_v.at[nxt]], bufs[k], sems[k])

    return gk(table, idx3)


_MR = 256


def _conv_body(s_ref, w_ref, b_ref, o_ref):
    o_ref[...] = (
        jnp.dot(s_ref[...], w_ref[...], preferred_element_type=jnp.float32)
        + b_ref[...]
    )


def _conv_call(samples2d, w2d, b2d):
    rows, kc = samples2d.shape
    oc = w2d.shape[1]
    return pl.pallas_call(
        _conv_body,
        grid=(rows // _MR,),
        in_specs=[
            pl.BlockSpec((_MR, kc), lambda i: (i, 0)),
            pl.BlockSpec((kc, oc), lambda i: (0, 0)),
            pl.BlockSpec((1, oc), lambda i: (0, 0)),
        ],
        out_specs=pl.BlockSpec((_MR, oc), lambda i: (i, 0)),
        out_shape=jax.ShapeDtypeStruct((rows, oc), jnp.float32),
    )(samples2d, w2d, b2d)


def kernel(x, pcs, W, b):
    bb, c, n = x.shape
    oc = W.shape[0]

    pcsT = jnp.transpose(pcs, (0, 2, 1))
    xT = jnp.transpose(x, (0, 2, 1))
    info = plsc.get_sparse_core_info()
    nw = info.num_cores * info.num_subcores
    nit = (n * 8) // (nw * _GCHUNK)
    parts = []
    for bi in range(bb):
        sel = _sel_call(pcs[bi:bi + 1], pcsT[bi:bi + 1])
        table = jnp.concatenate(
            [xT[bi], jnp.zeros((1, c), jnp.float32)], axis=0)
        parts.append(_gather_call(table, sel.reshape(nw, nit, _GCHUNK)))
    samples = jnp.concatenate(parts, axis=0)

    w8 = jnp.concatenate([W[:, :, 0:1], W[:, :, 2:9]], axis=2)
    w2d = jnp.transpose(w8, (2, 1, 0)).reshape(8 * c, oc)
    out2d = _conv_call(samples.reshape(bb * n, 8 * c), w2d,
                       b.reshape(1, oc))
    return out2d.reshape(bb, n, oc).transpose(0, 2, 1)
